# Initial kernel scaffold; baseline (speedup 1.0000x reference)
import jax
import jax.numpy as jnp
from jax.experimental import pallas as pl


def kernel(src, q_w, q_b, k_w, k_b, sal_w, sal_b, st_w, st_b, out_w, out_b, ln1_g, ln1_b, ln2_g, ln2_b, gate_w, gate_b, e_w1, e_b1, e_ow, e_ob, e_pw, e_pb, e_w2, e_b2):
    raise NotImplementedError("write your pallas kernel here")


# speedup vs baseline: 1.9225x; 1.9225x over previous
import math

import jax
import jax.numpy as jnp
from jax.experimental import pallas as pl
from jax.experimental.pallas import tpu as pltpu

L, D = 2048, 768
DC = D // 2
H = 12
DH = DC // H
DFF = 1536
E = 8
BQ = 256
BT = 256
NT = L // BT
F32 = jnp.float32


def _dot(a, b):
    return jnp.dot(a, b, preferred_element_type=F32)


def _ka(x1_ref, qw_ref, qb_ref, kw_ref, kb_ref, w0_ref, w1_ref, w2_ref,
        sb_ref, q_out, k_out, sal_out):
    x1 = x1_ref[...]
    q_out[...] = _dot(x1, qw_ref[...]) + qb_ref[...]
    k_out[...] = _dot(x1, kw_ref[...]) + kb_ref[...]
    a0 = _dot(x1, w0_ref[...])
    a1 = _dot(x1, w1_ref[...])
    a2 = _dot(x1, w2_ref[...])
    z = jnp.zeros((1, H), F32)
    sal_out[...] = (jnp.concatenate([z, a0[:-1]], axis=0) + a1
                    + jnp.concatenate([a2[1:], z], axis=0) + sb_ref[...])


def _kb(q_ref, k_ref, sal_ref, attw_ref, ctx_ref):
    q = q_ref[0]
    k = k_ref[0]
    s = _dot(q, k.T) * (1.0 / math.sqrt(DH)) + sal_ref[0]
    m = jnp.max(s, axis=-1, keepdims=True)
    p = jnp.exp(s - m)
    attw = p / jnp.sum(p, axis=-1, keepdims=True)
    attw_ref[0] = attw
    ctx_ref[0] = _dot(attw, k)


def _kc(ctx_ref, x1_ref, x2_ref, src_ref, stw_ref, stb_ref, w1_ref, w2_ref,
        ob_ref, l1g_ref, l1b_ref, gw_ref, gb_ref, h1_ref, cmb_ref, mean_ref):
    gate = jax.nn.sigmoid(_dot(ctx_ref[...], stw_ref[...]) + stb_ref[...])
    x2 = x2_ref[...]
    y2 = gate * jnp.tanh(x2) + (1.0 - gate) * x2
    attn = _dot(x1_ref[...], w1_ref[...]) + _dot(y2, w2_ref[...]) + ob_ref[...]
    h = src_ref[...] + attn
    mu = jnp.mean(h, axis=-1, keepdims=True)
    var = jnp.mean((h - mu) ** 2, axis=-1, keepdims=True)
    h1 = (h - mu) / jnp.sqrt(var + 1e-5) * l1g_ref[...] + l1b_ref[...]
    h1_ref[...] = h1
    logits = _dot(h1, gw_ref[...]) + gb_ref[...]
    lm = jnp.max(logits, axis=-1, keepdims=True)
    ex = jnp.exp(logits - lm)
    gates = ex / jnp.sum(ex, axis=-1, keepdims=True)
    iot = jax.lax.broadcasted_iota(jnp.int32, (L, E), 1)
    v1 = jnp.max(gates, axis=-1, keepdims=True)
    i1 = jnp.min(jnp.where(gates == v1, iot, E), axis=-1, keepdims=True)
    masked = jnp.where(iot == i1, -1.0, gates)
    v2 = jnp.max(masked, axis=-1, keepdims=True)
    i2 = jnp.min(jnp.where(masked == v2, iot, E), axis=-1, keepdims=True)
    oh1 = (iot == i1).astype(F32)
    oh2 = (iot == i2).astype(F32)
    cmb_ref[...] = (oh1 * v1 + oh2 * v2) / (v1 + v2)
    m = oh1 + oh2
    msum = jax.lax.dot_general(m, h1, (((0,), (0,)), ((), ())),
                               preferred_element_type=F32)
    cnt = jax.lax.dot_general(m, jnp.ones((L, 1), F32),
                              (((0,), (0,)), ((), ())),
                              preferred_element_type=F32)
    mean_ref[...] = msum / jnp.maximum(cnt, 1.0)


def _softplus(x):
    return jnp.maximum(x, 0.0) + jnp.log1p(jnp.exp(-jnp.abs(x)))


def _kd(mean_ref, ew1_ref, eb1_ref, eow_ref, eob_ref, epw_ref, epb_ref,
        om_ref, ph_ref):
    stats = _dot(mean_ref[0], ew1_ref[0]) + eb1_ref[0]
    a = _dot(stats, eow_ref[0]) + eob_ref[0]
    om_ref[0] = _softplus(a)
    ph_ref[0] = _dot(stats, epw_ref[0]) + epb_ref[0]


def _gelu(x):
    return 0.5 * x * (1.0 + jax.lax.erf(x * (1.0 / math.sqrt(2.0))))


def _ke(h1_ref, cmb_ref, om_ref, ph_ref, ew1_ref, eb1_ref, ew2_ref, eb2_ref,
        l2g_ref, l2b_ref, out_ref, acc_ref):
    e = pl.program_id(0)
    t = pl.program_id(1)
    hh = _dot(h1_ref[...], ew1_ref[0]) + eb1_ref[0]
    a = om_ref[0] * hh + ph_ref[0]
    act = _gelu(a)
    iot = jax.lax.broadcasted_iota(jnp.int32, (BT, E), 1)
    w = jnp.sum(jnp.where(iot == e, cmb_ref[...], 0.0), axis=1, keepdims=True)
    contrib = (_dot(act, ew2_ref[0]) + eb2_ref[0]) * w
    sl = pl.ds(t * BT, BT)

    @pl.when(e == 0)
    def _():
        acc_ref[sl, :] = contrib

    @pl.when(e > 0)
    def _():
        acc_ref[sl, :] = acc_ref[sl, :] + contrib

    @pl.when(e == E - 1)
    def _():
        h = h1_ref[...] + acc_ref[sl, :]
        mu = jnp.mean(h, axis=-1, keepdims=True)
        var = jnp.mean((h - mu) ** 2, axis=-1, keepdims=True)
        out_ref[...] = (h - mu) / jnp.sqrt(var + 1e-5) * l2g_ref[...] + l2b_ref[...]


def kernel(src, q_w, q_b, k_w, k_b, sal_w, sal_b, st_w, st_b, out_w, out_b,
           ln1_g, ln1_b, ln2_g, ln2_b, gate_w, gate_b,
           e_w1, e_b1, e_ow, e_ob, e_pw, e_pb, e_w2, e_b2):
    x = src[0]
    x1 = x[:, :DC]
    x2 = x[:, DC:]

    f32 = lambda s: jax.ShapeDtypeStruct(s, F32)

    Q, K, sal = pl.pallas_call(
        _ka,
        out_shape=[f32((L, DC)), f32((L, DC)), f32((L, H))],
    )(x1, q_w, q_b[None], k_w, k_b[None],
      sal_w[:, :, 0].T, sal_w[:, :, 1].T, sal_w[:, :, 2].T, sal_b[None])

    Qh = Q.reshape(L, H, DH).transpose(1, 0, 2)
    Kh = K.reshape(L, H, DH).transpose(1, 0, 2)
    salh = sal.T[:, None, :]

    attw, ctx = pl.pallas_call(
        _kb,
        grid=(H, L // BQ),
        in_specs=[
            pl.BlockSpec((1, BQ, DH), lambda h, q: (h, q, 0)),
            pl.BlockSpec((1, L, DH), lambda h, q: (h, 0, 0)),
            pl.BlockSpec((1, 1, L), lambda h, q: (h, 0, 0)),
        ],
        out_specs=[
            pl.BlockSpec((1, BQ, L), lambda h, q: (h, q, 0)),
            pl.BlockSpec((1, BQ, DH), lambda h, q: (h, q, 0)),
        ],
        out_shape=[f32((H, L, L)), f32((H, L, DH))],
    )(Qh, Kh, salh)

    ctxf = ctx.transpose(1, 0, 2).reshape(L, DC)

    h1, cmb, mean = pl.pallas_call(
        _kc,
        out_shape=[f32((L, D)), f32((L, E)), f32((E, D))],
    )(ctxf, x1, x2, x, st_w, st_b[None], out_w[:DC], out_w[DC:], out_b[None],
      ln1_g[None], ln1_b[None], gate_w, gate_b[None])

    omega, phi = pl.pallas_call(
        _kd,
        grid=(E,),
        in_specs=[
            pl.BlockSpec((1, 1, D), lambda e: (e, 0, 0)),
            pl.BlockSpec((1, D, DFF), lambda e: (e, 0, 0)),
            pl.BlockSpec((1, 1, DFF), lambda e: (e, 0, 0)),
            pl.BlockSpec((1, DFF, DFF), lambda e: (e, 0, 0)),
            pl.BlockSpec((1, 1, DFF), lambda e: (e, 0, 0)),
            pl.BlockSpec((1, DFF, DFF), lambda e: (e, 0, 0)),
            pl.BlockSpec((1, 1, DFF), lambda e: (e, 0, 0)),
        ],
        out_specs=[
            pl.BlockSpec((1, 1, DFF), lambda e: (e, 0, 0)),
            pl.BlockSpec((1, 1, DFF), lambda e: (e, 0, 0)),
        ],
        out_shape=[f32((E, 1, DFF)), f32((E, 1, DFF))],
    )(mean[:, None, :], e_w1, e_b1[:, None, :], e_ow, e_ob[:, None, :],
      e_pw, e_pb[:, None, :])

    out2d = pl.pallas_call(
        _ke,
        grid=(E, NT),
        in_specs=[
            pl.BlockSpec((BT, D), lambda e, t: (t, 0)),
            pl.BlockSpec((BT, E), lambda e, t: (t, 0)),
            pl.BlockSpec((1, 1, DFF), lambda e, t: (e, 0, 0)),
            pl.BlockSpec((1, 1, DFF), lambda e, t: (e, 0, 0)),
            pl.BlockSpec((1, D, DFF), lambda e, t: (e, 0, 0)),
            pl.BlockSpec((1, 1, DFF), lambda e, t: (e, 0, 0)),
            pl.BlockSpec((1, DFF, D), lambda e, t: (e, 0, 0)),
            pl.BlockSpec((1, 1, D), lambda e, t: (e, 0, 0)),
            pl.BlockSpec((1, D), lambda e, t: (0, 0)),
            pl.BlockSpec((1, D), lambda e, t: (0, 0)),
        ],
        out_specs=pl.BlockSpec((BT, D), lambda e, t: (t, 0)),
        out_shape=f32((L, D)),
        scratch_shapes=[pltpu.VMEM((L, D), F32)],
    )(h1, cmb, omega, phi, e_w1, e_b1[:, None, :], e_w2, e_b2[:, None, :],
      ln2_g[None], ln2_b[None])

    return out2d[None], attw[None]


# speedup vs baseline: 2.0086x; 1.0448x over previous
import math

import jax
import jax.numpy as jnp
from jax.experimental import pallas as pl
from jax.experimental.pallas import tpu as pltpu

L, D = 2048, 768
DC = D // 2
H = 12
DH = DC // H
DFF = 1536
E = 8
BQ = 256
BT = 256
NT = L // BT
F32 = jnp.float32


def _dot(a, b):
    return jnp.dot(a, b, preferred_element_type=F32)


BF16 = jnp.bfloat16


def _bdot(a, b):
    return jnp.dot(a.astype(BF16), b.astype(BF16), preferred_element_type=F32)


def _ka(x1_ref, qw_ref, qb_ref, kw_ref, kb_ref, w0_ref, w1_ref, w2_ref,
        sb_ref, q_out, k_out, sal_out):
    x1 = x1_ref[...]
    q_out[...] = _bdot(x1, qw_ref[...]) + qb_ref[...]
    k_out[...] = _bdot(x1, kw_ref[...]) + kb_ref[...]
    a0 = _bdot(x1, w0_ref[...])
    a1 = _bdot(x1, w1_ref[...])
    a2 = _bdot(x1, w2_ref[...])
    z = jnp.zeros((1, H), F32)
    sal_out[...] = (jnp.concatenate([z, a0[:-1]], axis=0) + a1
                    + jnp.concatenate([a2[1:], z], axis=0) + sb_ref[...])


def _kb(q_ref, k_ref, sal_ref, attw_ref, ctx_ref):
    q = q_ref[0]
    k = k_ref[0]
    s = _bdot(q, k.T) * (1.0 / math.sqrt(DH)) + sal_ref[0]
    m = jnp.max(s, axis=-1, keepdims=True)
    p = jnp.exp(s - m)
    attw = p / jnp.sum(p, axis=-1, keepdims=True)
    attw_ref[0] = attw
    ctx_ref[0] = _bdot(attw, k)


def _kc(ctx_ref, x1_ref, x2_ref, src_ref, stw_ref, stb_ref, w1_ref, w2_ref,
        ob_ref, l1g_ref, l1b_ref, gw_ref, gb_ref, h1_ref, cmb_ref, mean_ref):
    gate = jax.nn.sigmoid(_bdot(ctx_ref[...], stw_ref[...]) + stb_ref[...])
    x2 = x2_ref[...]
    y2 = gate * jnp.tanh(x2) + (1.0 - gate) * x2
    attn = _bdot(x1_ref[...], w1_ref[...]) + _bdot(y2, w2_ref[...]) + ob_ref[...]
    h = src_ref[...] + attn
    mu = jnp.mean(h, axis=-1, keepdims=True)
    var = jnp.mean((h - mu) ** 2, axis=-1, keepdims=True)
    h1 = (h - mu) / jnp.sqrt(var + 1e-5) * l1g_ref[...] + l1b_ref[...]
    h1_ref[...] = h1
    logits = _dot(h1, gw_ref[...]) + gb_ref[...]
    lm = jnp.max(logits, axis=-1, keepdims=True)
    ex = jnp.exp(logits - lm)
    gates = ex / jnp.sum(ex, axis=-1, keepdims=True)
    iot = jax.lax.broadcasted_iota(jnp.int32, (L, E), 1)
    v1 = jnp.max(gates, axis=-1, keepdims=True)
    i1 = jnp.min(jnp.where(gates == v1, iot, E), axis=-1, keepdims=True)
    masked = jnp.where(iot == i1, -1.0, gates)
    v2 = jnp.max(masked, axis=-1, keepdims=True)
    i2 = jnp.min(jnp.where(masked == v2, iot, E), axis=-1, keepdims=True)
    oh1 = (iot == i1).astype(F32)
    oh2 = (iot == i2).astype(F32)
    cmb_ref[...] = (oh1 * v1 + oh2 * v2) / (v1 + v2)
    m = oh1 + oh2
    msum = jax.lax.dot_general(m, h1, (((0,), (0,)), ((), ())),
                               preferred_element_type=F32)
    cnt = jax.lax.dot_general(m, jnp.ones((L, 1), F32),
                              (((0,), (0,)), ((), ())),
                              preferred_element_type=F32)
    mean_ref[...] = msum / jnp.maximum(cnt, 1.0)


def _softplus(x):
    return jnp.maximum(x, 0.0) + jnp.log1p(jnp.exp(-jnp.abs(x)))


def _kd(mean_ref, ew1_ref, eb1_ref, eow_ref, eob_ref, epw_ref, epb_ref,
        om_ref, ph_ref):
    stats = _dot(mean_ref[0], ew1_ref[0]) + eb1_ref[0]
    a = _dot(stats, eow_ref[0]) + eob_ref[0]
    om_ref[0] = _softplus(a)
    ph_ref[0] = _dot(stats, epw_ref[0]) + epb_ref[0]


def _gelu(x):
    return 0.5 * x * (1.0 + jax.lax.erf(x * (1.0 / math.sqrt(2.0))))


def _ke(h1_ref, cmb_ref, om_ref, ph_ref, ew1_ref, eb1_ref, ew2_ref, eb2_ref,
        l2g_ref, l2b_ref, out_ref, acc_ref):
    e = pl.program_id(0)
    t = pl.program_id(1)
    hh = _bdot(h1_ref[...], ew1_ref[0]) + eb1_ref[0]
    a = om_ref[0] * hh + ph_ref[0]
    act = _gelu(a)
    iot = jax.lax.broadcasted_iota(jnp.int32, (BT, E), 1)
    w = jnp.sum(jnp.where(iot == e, cmb_ref[...], 0.0), axis=1, keepdims=True)
    contrib = (_bdot(act, ew2_ref[0]) + eb2_ref[0]) * w
    sl = pl.ds(t * BT, BT)

    @pl.when(e == 0)
    def _():
        acc_ref[sl, :] = contrib

    @pl.when(e > 0)
    def _():
        acc_ref[sl, :] = acc_ref[sl, :] + contrib

    @pl.when(e == E - 1)
    def _():
        h = h1_ref[...] + acc_ref[sl, :]
        mu = jnp.mean(h, axis=-1, keepdims=True)
        var = jnp.mean((h - mu) ** 2, axis=-1, keepdims=True)
        out_ref[...] = (h - mu) / jnp.sqrt(var + 1e-5) * l2g_ref[...] + l2b_ref[...]


def kernel(src, q_w, q_b, k_w, k_b, sal_w, sal_b, st_w, st_b, out_w, out_b,
           ln1_g, ln1_b, ln2_g, ln2_b, gate_w, gate_b,
           e_w1, e_b1, e_ow, e_ob, e_pw, e_pb, e_w2, e_b2):
    x = src[0]
    x1 = x[:, :DC]
    x2 = x[:, DC:]

    f32 = lambda s: jax.ShapeDtypeStruct(s, F32)

    Q, K, sal = pl.pallas_call(
        _ka,
        out_shape=[f32((L, DC)), f32((L, DC)), f32((L, H))],
    )(x1, q_w, q_b[None], k_w, k_b[None],
      sal_w[:, :, 0].T, sal_w[:, :, 1].T, sal_w[:, :, 2].T, sal_b[None])

    Qh = Q.reshape(L, H, DH).transpose(1, 0, 2)
    Kh = K.reshape(L, H, DH).transpose(1, 0, 2)
    salh = sal.T[:, None, :]

    attw, ctx = pl.pallas_call(
        _kb,
        grid=(H, L // BQ),
        in_specs=[
            pl.BlockSpec((1, BQ, DH), lambda h, q: (h, q, 0)),
            pl.BlockSpec((1, L, DH), lambda h, q: (h, 0, 0)),
            pl.BlockSpec((1, 1, L), lambda h, q: (h, 0, 0)),
        ],
        out_specs=[
            pl.BlockSpec((1, BQ, L), lambda h, q: (h, q, 0)),
            pl.BlockSpec((1, BQ, DH), lambda h, q: (h, q, 0)),
        ],
        out_shape=[f32((H, L, L)), f32((H, L, DH))],
    )(Qh, Kh, salh)

    ctxf = ctx.transpose(1, 0, 2).reshape(L, DC)

    h1, cmb, mean = pl.pallas_call(
        _kc,
        out_shape=[f32((L, D)), f32((L, E)), f32((E, D))],
    )(ctxf, x1, x2, x, st_w, st_b[None], out_w[:DC], out_w[DC:], out_b[None],
      ln1_g[None], ln1_b[None], gate_w, gate_b[None])

    omega, phi = pl.pallas_call(
        _kd,
        grid=(E,),
        in_specs=[
            pl.BlockSpec((1, 1, D), lambda e: (e, 0, 0)),
            pl.BlockSpec((1, D, DFF), lambda e: (e, 0, 0)),
            pl.BlockSpec((1, 1, DFF), lambda e: (e, 0, 0)),
            pl.BlockSpec((1, DFF, DFF), lambda e: (e, 0, 0)),
            pl.BlockSpec((1, 1, DFF), lambda e: (e, 0, 0)),
            pl.BlockSpec((1, DFF, DFF), lambda e: (e, 0, 0)),
            pl.BlockSpec((1, 1, DFF), lambda e: (e, 0, 0)),
        ],
        out_specs=[
            pl.BlockSpec((1, 1, DFF), lambda e: (e, 0, 0)),
            pl.BlockSpec((1, 1, DFF), lambda e: (e, 0, 0)),
        ],
        out_shape=[f32((E, 1, DFF)), f32((E, 1, DFF))],
    )(mean[:, None, :], e_w1, e_b1[:, None, :], e_ow, e_ob[:, None, :],
      e_pw, e_pb[:, None, :])

    out2d = pl.pallas_call(
        _ke,
        grid=(E, NT),
        in_specs=[
            pl.BlockSpec((BT, D), lambda e, t: (t, 0)),
            pl.BlockSpec((BT, E), lambda e, t: (t, 0)),
            pl.BlockSpec((1, 1, DFF), lambda e, t: (e, 0, 0)),
            pl.BlockSpec((1, 1, DFF), lambda e, t: (e, 0, 0)),
            pl.BlockSpec((1, D, DFF), lambda e, t: (e, 0, 0)),
            pl.BlockSpec((1, 1, DFF), lambda e, t: (e, 0, 0)),
            pl.BlockSpec((1, DFF, D), lambda e, t: (e, 0, 0)),
            pl.BlockSpec((1, 1, D), lambda e, t: (e, 0, 0)),
            pl.BlockSpec((1, D), lambda e, t: (0, 0)),
            pl.BlockSpec((1, D), lambda e, t: (0, 0)),
        ],
        out_specs=pl.BlockSpec((BT, D), lambda e, t: (t, 0)),
        out_shape=f32((L, D)),
        scratch_shapes=[pltpu.VMEM((L, D), F32)],
    )(h1, cmb, omega, phi, e_w1, e_b1[:, None, :], e_w2, e_b2[:, None, :],
      ln2_g[None], ln2_b[None])

    return out2d[None], attw[None]
